# baseline (speedup 1.0000x reference)
import jax
import jax.numpy as jnp
from jax.experimental import pallas as pl
from jax.experimental.pallas import tpu as pltpu

S = 4096
M = 1024
GE = 64
CAP = 128
NW = GE * CAP // 32
BS = 256
NBLK = S // BS


def _gate_kernel(x_ref, wt_ref, cw_ref, mask_ref, loss_ref, counts_ref, me_ref):
    pid = pl.program_id(0)

    @pl.when(pid == 0)
    def _init():
        counts_ref[...] = jnp.zeros_like(counts_ref)
        me_ref[...] = jnp.zeros_like(me_ref)

    x = x_ref[...]
    wt = wt_ref[...]
    logits = jnp.dot(x, wt, preferred_element_type=jnp.float32)

    row_max = jnp.max(logits, axis=1, keepdims=True)
    p = jnp.exp(logits - row_max)
    gates = p / jnp.sum(p, axis=1, keepdims=True)

    eids = jax.lax.broadcasted_iota(jnp.int32, (BS, GE), 1)
    eidx = jnp.min(jnp.where(logits == row_max, eids, GE), axis=1, keepdims=True)
    onehot_f = (eids == eidx).astype(jnp.float32)

    r = jax.lax.broadcasted_iota(jnp.int32, (BS, BS), 0)
    c = jax.lax.broadcasted_iota(jnp.int32, (BS, BS), 1)
    ltri = (r >= c).astype(jnp.float32)
    incl = jnp.dot(ltri, onehot_f, preferred_element_type=jnp.float32)

    counts = counts_ref[...]
    loc_s = jnp.sum((incl - 1.0 + counts) * onehot_f, axis=1, keepdims=True)
    kept = loc_s < CAP
    gate_val = jnp.sum(gates * onehot_f, axis=1, keepdims=True)

    pos = eidx * CAP + loc_s.astype(jnp.int32)
    pos = jnp.where(kept, pos, -1)

    fi = jax.lax.broadcasted_iota(jnp.int32, (BS, GE, CAP), 1) * CAP + \
        jax.lax.broadcasted_iota(jnp.int32, (BS, GE, CAP), 2)
    cond = fi == pos[:, :, None]
    cw_ref[...] = jnp.where(cond, gate_val[:, :, None], 0.0)

    wpos = jnp.where(kept, pos >> 5, -1)
    wval = jnp.left_shift(jnp.int32(1), pos & 31)
    wi = jax.lax.broadcasted_iota(jnp.int32, (BS, NW), 1)
    mask_ref[...] = jnp.where(wi == wpos, wval, 0)

    counts_ref[...] = counts + jnp.sum(onehot_f, axis=0, keepdims=True)
    me_ref[...] = me_ref[...] + jnp.sum(gates, axis=0, keepdims=True)

    @pl.when(pid == NBLK - 1)
    def _fin():
        loss_ref[...] = jnp.sum(
            me_ref[...] * counts_ref[...], axis=(0, 1), keepdims=True
        ) * (GE / (S * S))


def kernel(in_data, W):
    wt = W.T
    cw, mask8, loss = pl.pallas_call(
        _gate_kernel,
        grid=(NBLK,),
        in_specs=[
            pl.BlockSpec((BS, M), lambda i: (i, 0)),
            pl.BlockSpec((M, GE), lambda i: (0, 0)),
        ],
        out_specs=[
            pl.BlockSpec((BS, GE, CAP), lambda i: (i, 0, 0)),
            pl.BlockSpec((BS, NW), lambda i: (i, 0)),
            pl.BlockSpec((1, 1), lambda i: (0, 0)),
        ],
        out_shape=[
            jax.ShapeDtypeStruct((S, GE, CAP), jnp.float32),
            jax.ShapeDtypeStruct((S, NW), jnp.int32),
            jax.ShapeDtypeStruct((1, 1), jnp.float32),
        ],
        scratch_shapes=[
            pltpu.VMEM((1, GE), jnp.float32),
            pltpu.VMEM((1, GE), jnp.float32),
        ],
    )(in_data, wt)
    bitmasks = jnp.left_shift(jnp.int32(1), jnp.arange(32, dtype=jnp.int32))
    mask = ((mask8[:, :, None] & bitmasks) != 0).reshape(S, GE, CAP)
    return (cw, mask, loss[0, 0])


# speedup vs baseline: 1.9611x; 1.9611x over previous
import jax
import jax.numpy as jnp
from jax.experimental import pallas as pl
from jax.experimental.pallas import tpu as pltpu

S = 4096
M = 1024
GE = 64
CAP = 128
BS = 256
NBLK = S // BS
WBS = 256
NWBLK = S // WBS


def _route_kernel(x_ref, wt_ref, pos_ref, gv_ref, loss_ref, counts_ref, me_ref):
    pid = pl.program_id(0)

    @pl.when(pid == 0)
    def _init():
        counts_ref[...] = jnp.zeros_like(counts_ref)
        me_ref[...] = jnp.zeros_like(me_ref)

    x = x_ref[...]
    wt = wt_ref[...]
    logits = jnp.dot(x, wt, preferred_element_type=jnp.float32)

    row_max = jnp.max(logits, axis=1, keepdims=True)
    p = jnp.exp(logits - row_max)
    gates = p / jnp.sum(p, axis=1, keepdims=True)

    eids = jax.lax.broadcasted_iota(jnp.int32, (BS, GE), 1)
    eidx = jnp.min(jnp.where(logits == row_max, eids, GE), axis=1, keepdims=True)
    onehot_f = (eids == eidx).astype(jnp.float32)

    r = jax.lax.broadcasted_iota(jnp.int32, (BS, BS), 0)
    c = jax.lax.broadcasted_iota(jnp.int32, (BS, BS), 1)
    ltri = (r >= c).astype(jnp.float32)
    incl = jnp.dot(ltri, onehot_f, preferred_element_type=jnp.float32)

    counts = counts_ref[...]
    loc_s = jnp.sum((incl - 1.0 + counts) * onehot_f, axis=1, keepdims=True)
    kept = loc_s < CAP
    gate_val = jnp.sum(gates * onehot_f, axis=1, keepdims=True)

    pos = eidx * CAP + loc_s.astype(jnp.int32)
    pos_ref[...] = jnp.where(kept, pos, -1)
    gv_ref[...] = gate_val

    counts_ref[...] = counts + jnp.sum(onehot_f, axis=0, keepdims=True)
    me_ref[...] = me_ref[...] + jnp.sum(gates, axis=0, keepdims=True)

    @pl.when(pid == NBLK - 1)
    def _fin():
        loss_ref[...] = jnp.sum(
            me_ref[...] * counts_ref[...], axis=(0, 1), keepdims=True
        ) * (GE / (S * S))


def _write_kernel(pos_ref, gv_ref, cw_ref, mask_ref):
    pos = pos_ref[...]
    gv = gv_ref[...]
    fi = jax.lax.broadcasted_iota(jnp.int32, (WBS, GE, CAP), 1) * CAP + \
        jax.lax.broadcasted_iota(jnp.int32, (WBS, GE, CAP), 2)
    cond = fi == pos[:, :, None]
    cw_ref[...] = jnp.where(cond, gv[:, :, None], 0.0)
    mask_ref[...] = cond.astype(jnp.int8)


def kernel(in_data, W):
    wt = W.T
    pos, gv, loss = pl.pallas_call(
        _route_kernel,
        grid=(NBLK,),
        in_specs=[
            pl.BlockSpec((BS, M), lambda i: (i, 0)),
            pl.BlockSpec((M, GE), lambda i: (0, 0)),
        ],
        out_specs=[
            pl.BlockSpec((BS, 1), lambda i: (i, 0)),
            pl.BlockSpec((BS, 1), lambda i: (i, 0)),
            pl.BlockSpec((1, 1), lambda i: (0, 0)),
        ],
        out_shape=[
            jax.ShapeDtypeStruct((S, 1), jnp.int32),
            jax.ShapeDtypeStruct((S, 1), jnp.float32),
            jax.ShapeDtypeStruct((1, 1), jnp.float32),
        ],
        scratch_shapes=[
            pltpu.VMEM((1, GE), jnp.float32),
            pltpu.VMEM((1, GE), jnp.float32),
        ],
    )(in_data, wt)
    cw, mask8 = pl.pallas_call(
        _write_kernel,
        grid=(NWBLK,),
        in_specs=[
            pl.BlockSpec((WBS, 1), lambda i: (i, 0)),
            pl.BlockSpec((WBS, 1), lambda i: (i, 0)),
        ],
        out_specs=[
            pl.BlockSpec((WBS, GE, CAP), lambda i: (i, 0, 0)),
            pl.BlockSpec((WBS, GE, CAP), lambda i: (i, 0, 0)),
        ],
        out_shape=[
            jax.ShapeDtypeStruct((S, GE, CAP), jnp.float32),
            jax.ShapeDtypeStruct((S, GE, CAP), jnp.int8),
        ],
    )(pos, gv)
    return (cw, mask8.astype(jnp.bool_), loss[0, 0])


# speedup vs baseline: 2.2878x; 1.1666x over previous
import jax
import jax.numpy as jnp
from jax.experimental import pallas as pl
from jax.experimental.pallas import tpu as pltpu

S = 4096
M = 1024
GE = 64
CAP = 128
BS = 256
NBLK = S // BS


def _gate_kernel(x_ref, wt_ref, cw_ref, mask_ref, loss_ref, counts_ref, me_ref):
    pid = pl.program_id(0)

    @pl.when(pid == 0)
    def _init():
        counts_ref[...] = jnp.zeros_like(counts_ref)
        me_ref[...] = jnp.zeros_like(me_ref)

    x = x_ref[...]
    wt = wt_ref[...]
    logits = jnp.dot(x, wt, preferred_element_type=jnp.float32)

    row_max = jnp.max(logits, axis=1, keepdims=True)
    p = jnp.exp(logits - row_max)
    gates = p / jnp.sum(p, axis=1, keepdims=True)

    eids = jax.lax.broadcasted_iota(jnp.int32, (BS, GE), 1)
    eidx = jnp.min(jnp.where(logits == row_max, eids, GE), axis=1, keepdims=True)
    onehot_f = (eids == eidx).astype(jnp.float32)

    r = jax.lax.broadcasted_iota(jnp.int32, (BS, BS), 0)
    c = jax.lax.broadcasted_iota(jnp.int32, (BS, BS), 1)
    ltri = (r >= c).astype(jnp.float32)
    incl = jnp.dot(ltri, onehot_f, preferred_element_type=jnp.float32)

    counts = counts_ref[...]
    loc_s = jnp.sum((incl - 1.0 + counts) * onehot_f, axis=1, keepdims=True)
    kept = loc_s < CAP
    gate_val = jnp.sum(gates * onehot_f, axis=1, keepdims=True)

    pos = eidx * CAP + loc_s.astype(jnp.int32)
    pos = jnp.where(kept, pos, -1)

    fi = jax.lax.broadcasted_iota(jnp.int32, (BS, GE, CAP), 1) * CAP + \
        jax.lax.broadcasted_iota(jnp.int32, (BS, GE, CAP), 2)
    cond = fi == pos[:, :, None]
    cw_ref[...] = jnp.where(cond, gate_val[:, :, None], 0.0)
    mask_ref[...] = cond.astype(jnp.int8)

    counts_ref[...] = counts + jnp.sum(onehot_f, axis=0, keepdims=True)
    me_ref[...] = me_ref[...] + jnp.sum(gates, axis=0, keepdims=True)

    @pl.when(pid == NBLK - 1)
    def _fin():
        loss_ref[...] = jnp.sum(
            me_ref[...] * counts_ref[...], axis=(0, 1), keepdims=True
        ) * (GE / (S * S))


def kernel(in_data, W):
    wt = W.T
    cw, mask8, loss = pl.pallas_call(
        _gate_kernel,
        grid=(NBLK,),
        in_specs=[
            pl.BlockSpec((BS, M), lambda i: (i, 0)),
            pl.BlockSpec((M, GE), lambda i: (0, 0)),
        ],
        out_specs=[
            pl.BlockSpec((BS, GE, CAP), lambda i: (i, 0, 0)),
            pl.BlockSpec((BS, GE, CAP), lambda i: (i, 0, 0)),
            pl.BlockSpec((1, 1), lambda i: (0, 0)),
        ],
        out_shape=[
            jax.ShapeDtypeStruct((S, GE, CAP), jnp.float32),
            jax.ShapeDtypeStruct((S, GE, CAP), jnp.int8),
            jax.ShapeDtypeStruct((1, 1), jnp.float32),
        ],
        scratch_shapes=[
            pltpu.VMEM((1, GE), jnp.float32),
            pltpu.VMEM((1, GE), jnp.float32),
        ],
    )(in_data, wt)
    return (cw, mask8.astype(jnp.bool_), loss[0, 0])
